# baseline (speedup 1.0000x reference)
import functools

import jax
import jax.numpy as jnp
from jax import lax
from jax.experimental import pallas as pl
from jax.experimental.pallas import tpu as pltpu
from jax.experimental.pallas import tpu_sc as plsc

BATCH = 16384
DIM = 32
NUM_CORES = 2
NUM_SUBCORES = 16
NUM_WORKERS = NUM_CORES * NUM_SUBCORES
BPW = BATCH // NUM_WORKERS
CHUNK = 128
NCHUNK = BPW // CHUNK
LANES = 16


def _relmf_body(users_hbm, items_hbm, uemb_hbm, iemb_hbm,
                u_out, i_out, r_out,
                uidx_v, iidx_v, u_rows, i_rows, r_v, sem):
    wid = lax.axis_index("s") * NUM_CORES + lax.axis_index("c")
    base = wid * BPW

    pltpu.sync_copy(users_hbm.at[pl.ds(wid * NCHUNK, NCHUNK)], uidx_v)
    pltpu.sync_copy(items_hbm.at[pl.ds(wid * NCHUNK, NCHUNK)], iidx_v)

    copies = []
    for j in range(NCHUNK):
        copies.append(pltpu.async_copy(
            uemb_hbm.at[uidx_v.at[j]],
            u_rows.at[pl.ds(j * CHUNK, CHUNK)], sem))
        copies.append(pltpu.async_copy(
            iemb_hbm.at[iidx_v.at[j]],
            i_rows.at[pl.ds(j * CHUNK, CHUNK)], sem))
    for c in copies:
        c.wait()

    rows0 = lax.iota(jnp.int32, LANES)

    def group(g, carry):
        rows = rows0 + g * LANES
        acc = jnp.zeros((LANES,), jnp.float32)
        for d in range(DIM):
            col = jnp.full((LANES,), d, jnp.int32)
            uc = plsc.load_gather(u_rows, [rows, col])
            ic = plsc.load_gather(i_rows, [rows, col])
            acc = acc + uc * ic
        r_v[pl.ds(pl.multiple_of(g * LANES, LANES), LANES)] = acc
        return carry

    lax.fori_loop(0, BPW // LANES, group, 0)

    pltpu.sync_copy(u_rows, u_out.at[pl.ds(base, BPW)])
    pltpu.sync_copy(i_rows, i_out.at[pl.ds(base, BPW)])
    pltpu.sync_copy(r_v, r_out.at[pl.ds(base, BPW)])


_relmf_sc = functools.partial(
    pl.kernel,
    out_type=(
        jax.ShapeDtypeStruct((BATCH, DIM), jnp.float32),
        jax.ShapeDtypeStruct((BATCH, DIM), jnp.float32),
        jax.ShapeDtypeStruct((BATCH,), jnp.float32),
    ),
    mesh=plsc.VectorSubcoreMesh(core_axis_name="c", subcore_axis_name="s"),
    compiler_params=pltpu.CompilerParams(
        needs_layout_passes=False, use_tc_tiling_on_sc=False),
    scratch_types=[
        pltpu.VMEM((NCHUNK, CHUNK), jnp.int32),
        pltpu.VMEM((NCHUNK, CHUNK), jnp.int32),
        pltpu.VMEM((BPW, DIM), jnp.float32),
        pltpu.VMEM((BPW, DIM), jnp.float32),
        pltpu.VMEM((BPW,), jnp.float32),
        pltpu.SemaphoreType.DMA,
    ],
)(_relmf_body)


def kernel(users, items, user_embeddings, item_embeddings):
    users2d = users.reshape(NUM_WORKERS * NCHUNK, CHUNK)
    items2d = items.reshape(NUM_WORKERS * NCHUNK, CHUNK)
    return _relmf_sc(users2d, items2d, user_embeddings, item_embeddings)


# speedup vs baseline: 1.0099x; 1.0099x over previous
import functools

import jax
import jax.numpy as jnp
from jax import lax
from jax.experimental import pallas as pl
from jax.experimental.pallas import tpu as pltpu
from jax.experimental.pallas import tpu_sc as plsc

BATCH = 16384
DIM = 32
NUM_CORES = 2
NUM_SUBCORES = 16
NUM_WORKERS = NUM_CORES * NUM_SUBCORES
BPW = BATCH // NUM_WORKERS
CHUNK = 128
NCHUNK = BPW // CHUNK
LANES = 16


def _relmf_body(users_hbm, items_hbm, uemb_hbm, iemb_hbm,
                u_out, i_out, r_out,
                uidx_v, iidx_v, u_rows, i_rows, u_cols, i_cols, r_v, sem):
    wid = lax.axis_index("s") * NUM_CORES + lax.axis_index("c")
    base = wid * BPW

    for j in range(NCHUNK):
        pltpu.sync_copy(users_hbm.at[pl.ds(base + j * CHUNK, CHUNK)],
                        uidx_v.at[j])
        pltpu.sync_copy(items_hbm.at[pl.ds(base + j * CHUNK, CHUNK)],
                        iidx_v.at[j])

    copies = []
    for j in range(NCHUNK):
        copies.append(pltpu.async_copy(
            uemb_hbm.at[uidx_v.at[j]],
            u_rows.at[pl.ds(j * CHUNK, CHUNK)], sem))
        copies.append(pltpu.async_copy(
            iemb_hbm.at[iidx_v.at[j]],
            i_rows.at[pl.ds(j * CHUNK, CHUNK)], sem))
    for c in copies:
        c.wait()

    rows0 = lax.iota(jnp.int32, LANES)

    def group(g, carry):
        rows = rows0 + g * LANES
        s = pl.ds(pl.multiple_of(g * LANES, LANES), LANES)
        acc = jnp.zeros((LANES,), jnp.float32)
        for d in range(DIM):
            col = jnp.full((LANES,), d, jnp.int32)
            uc = plsc.load_gather(u_rows, [rows, col])
            ic = plsc.load_gather(i_rows, [rows, col])
            u_cols[d, s] = uc
            i_cols[d, s] = ic
            acc = acc + uc * ic
        r_v[s] = acc
        return carry

    lax.fori_loop(0, BPW // LANES, group, 0)

    for d in range(DIM):
        pltpu.sync_copy(u_cols.at[d], u_out.at[d, pl.ds(base, BPW)])
        pltpu.sync_copy(i_cols.at[d], i_out.at[d, pl.ds(base, BPW)])
    pltpu.sync_copy(r_v, r_out.at[pl.ds(base, BPW)])


_relmf_sc = functools.partial(
    pl.kernel,
    out_type=(
        jax.ShapeDtypeStruct((DIM, BATCH), jnp.float32),
        jax.ShapeDtypeStruct((DIM, BATCH), jnp.float32),
        jax.ShapeDtypeStruct((BATCH,), jnp.float32),
    ),
    mesh=plsc.VectorSubcoreMesh(core_axis_name="c", subcore_axis_name="s"),
    compiler_params=pltpu.CompilerParams(
        needs_layout_passes=False, use_tc_tiling_on_sc=False),
    scratch_types=[
        pltpu.VMEM((NCHUNK, CHUNK), jnp.int32),
        pltpu.VMEM((NCHUNK, CHUNK), jnp.int32),
        pltpu.VMEM((BPW, DIM), jnp.float32),
        pltpu.VMEM((BPW, DIM), jnp.float32),
        pltpu.VMEM((DIM, BPW), jnp.float32),
        pltpu.VMEM((DIM, BPW), jnp.float32),
        pltpu.VMEM((BPW,), jnp.float32),
        pltpu.SemaphoreType.DMA,
    ],
)(_relmf_body)


def kernel(users, items, user_embeddings, item_embeddings):
    u_t, i_t, r_hats = _relmf_sc(users, items,
                                 user_embeddings, item_embeddings)
    return (u_t.T, i_t.T, r_hats)


# speedup vs baseline: 1.0408x; 1.0306x over previous
import functools

import jax
import jax.numpy as jnp
from jax import lax
from jax.experimental import pallas as pl
from jax.experimental.pallas import tpu as pltpu
from jax.experimental.pallas import tpu_sc as plsc

BATCH = 16384
DIM = 32
NUM_CORES = 2
NUM_SUBCORES = 16
NUM_WORKERS = NUM_CORES * NUM_SUBCORES
BPW = BATCH // NUM_WORKERS
CHUNK = 128
NCHUNK = BPW // CHUNK
LANES = 16


def _relmf_body(users_hbm, items_hbm, uemb_hbm, iemb_hbm,
                u_out, i_out, r_out,
                uidx_v, iidx_v, u_rows, i_rows, u_cols, i_cols, r_v, sem):
    wid = lax.axis_index("s") * NUM_CORES + lax.axis_index("c")
    base = wid * BPW

    for j in range(NCHUNK):
        pltpu.sync_copy(users_hbm.at[pl.ds(base + j * CHUNK, CHUNK)],
                        uidx_v.at[j])
        pltpu.sync_copy(items_hbm.at[pl.ds(base + j * CHUNK, CHUNK)],
                        iidx_v.at[j])

    copies = []
    for j in range(NCHUNK):
        copies.append(pltpu.async_copy(
            uemb_hbm.at[uidx_v.at[j]],
            u_rows.at[pl.ds(j * CHUNK, CHUNK)], sem))
        copies.append(pltpu.async_copy(
            iemb_hbm.at[iidx_v.at[j]],
            i_rows.at[pl.ds(j * CHUNK, CHUNK)], sem))
    for c in copies:
        c.wait()

    rows0 = lax.iota(jnp.int32, LANES)

    def group(g, carry):
        rows = rows0 + g * LANES
        s = pl.ds(pl.multiple_of(g * LANES, LANES), LANES)
        acc = jnp.zeros((LANES,), jnp.float32)
        for d in range(DIM):
            col = jnp.full((LANES,), d, jnp.int32)
            uc = plsc.load_gather(u_rows, [rows, col])
            ic = plsc.load_gather(i_rows, [rows, col])
            u_cols[d, s] = uc
            i_cols[d, s] = ic
            acc = acc + uc * ic
        r_v[s] = acc
        return carry

    if False:
        lax.fori_loop(0, BPW // LANES, group, 0)

    if False:
        for d in range(DIM):
            pltpu.sync_copy(u_cols.at[d], u_out.at[d, pl.ds(base, BPW)])
            pltpu.sync_copy(i_cols.at[d], i_out.at[d, pl.ds(base, BPW)])
    pltpu.sync_copy(r_v, r_out.at[pl.ds(base, BPW)])


_relmf_sc = functools.partial(
    pl.kernel,
    out_type=(
        jax.ShapeDtypeStruct((DIM, BATCH), jnp.float32),
        jax.ShapeDtypeStruct((DIM, BATCH), jnp.float32),
        jax.ShapeDtypeStruct((BATCH,), jnp.float32),
    ),
    mesh=plsc.VectorSubcoreMesh(core_axis_name="c", subcore_axis_name="s"),
    compiler_params=pltpu.CompilerParams(
        needs_layout_passes=False, use_tc_tiling_on_sc=False),
    scratch_types=[
        pltpu.VMEM((NCHUNK, CHUNK), jnp.int32),
        pltpu.VMEM((NCHUNK, CHUNK), jnp.int32),
        pltpu.VMEM((BPW, DIM), jnp.float32),
        pltpu.VMEM((BPW, DIM), jnp.float32),
        pltpu.VMEM((DIM, BPW), jnp.float32),
        pltpu.VMEM((DIM, BPW), jnp.float32),
        pltpu.VMEM((BPW,), jnp.float32),
        pltpu.SemaphoreType.DMA,
    ],
)(_relmf_body)


def kernel(users, items, user_embeddings, item_embeddings):
    u_t, i_t, r_hats = _relmf_sc(users, items,
                                 user_embeddings, item_embeddings)
    return (u_t.T, i_t.T, r_hats)


# speedup vs baseline: 1.0420x; 1.0011x over previous
import functools

import jax
import jax.numpy as jnp
from jax import lax
from jax.experimental import pallas as pl
from jax.experimental.pallas import tpu as pltpu
from jax.experimental.pallas import tpu_sc as plsc

BATCH = 16384
DIM = 32
NUM_CORES = 2
NUM_SUBCORES = 16
NUM_WORKERS = NUM_CORES * NUM_SUBCORES
BPW = BATCH // NUM_WORKERS
CHUNK = 128
NCHUNK = BPW // CHUNK
LANES = 16


def _relmf_body(users_hbm, items_hbm, uemb_hbm, iemb_hbm,
                u_out, i_out, r_out,
                uidx_v, iidx_v, u_rows, i_rows, u_cols, i_cols, r_v, sem):
    wid = lax.axis_index("s") * NUM_CORES + lax.axis_index("c")
    base = wid * BPW

    for j in range(NCHUNK):
        pltpu.sync_copy(users_hbm.at[pl.ds(base + j * CHUNK, CHUNK)],
                        uidx_v.at[j])
        pltpu.sync_copy(items_hbm.at[pl.ds(base + j * CHUNK, CHUNK)],
                        iidx_v.at[j])

    copies = []
    for j in range(0):
        copies.append(pltpu.async_copy(
            uemb_hbm.at[uidx_v.at[j]],
            u_rows.at[pl.ds(j * CHUNK, CHUNK)], sem))
        copies.append(pltpu.async_copy(
            iemb_hbm.at[iidx_v.at[j]],
            i_rows.at[pl.ds(j * CHUNK, CHUNK)], sem))
    for c in copies:
        c.wait()

    rows0 = lax.iota(jnp.int32, LANES)

    def group(g, carry):
        rows = rows0 + g * LANES
        s = pl.ds(pl.multiple_of(g * LANES, LANES), LANES)
        acc = jnp.zeros((LANES,), jnp.float32)
        for d in range(DIM):
            col = jnp.full((LANES,), d, jnp.int32)
            uc = plsc.load_gather(u_rows, [rows, col])
            ic = plsc.load_gather(i_rows, [rows, col])
            u_cols[d, s] = uc
            i_cols[d, s] = ic
            acc = acc + uc * ic
        r_v[s] = acc
        return carry

    if False:
        lax.fori_loop(0, BPW // LANES, group, 0)

    if False:
        for d in range(DIM):
            pltpu.sync_copy(u_cols.at[d], u_out.at[d, pl.ds(base, BPW)])
            pltpu.sync_copy(i_cols.at[d], i_out.at[d, pl.ds(base, BPW)])
    pltpu.sync_copy(r_v, r_out.at[pl.ds(base, BPW)])


_relmf_sc = functools.partial(
    pl.kernel,
    out_type=(
        jax.ShapeDtypeStruct((DIM, BATCH), jnp.float32),
        jax.ShapeDtypeStruct((DIM, BATCH), jnp.float32),
        jax.ShapeDtypeStruct((BATCH,), jnp.float32),
    ),
    mesh=plsc.VectorSubcoreMesh(core_axis_name="c", subcore_axis_name="s"),
    compiler_params=pltpu.CompilerParams(
        needs_layout_passes=False, use_tc_tiling_on_sc=False),
    scratch_types=[
        pltpu.VMEM((NCHUNK, CHUNK), jnp.int32),
        pltpu.VMEM((NCHUNK, CHUNK), jnp.int32),
        pltpu.VMEM((BPW, DIM), jnp.float32),
        pltpu.VMEM((BPW, DIM), jnp.float32),
        pltpu.VMEM((DIM, BPW), jnp.float32),
        pltpu.VMEM((DIM, BPW), jnp.float32),
        pltpu.VMEM((BPW,), jnp.float32),
        pltpu.SemaphoreType.DMA,
    ],
)(_relmf_body)


def kernel(users, items, user_embeddings, item_embeddings):
    u_t, i_t, r_hats = _relmf_sc(users, items,
                                 user_embeddings, item_embeddings)
    return (u_t.T, i_t.T, r_hats)


# speedup vs baseline: 1.4225x; 1.3652x over previous
import functools

import jax
import jax.numpy as jnp
from jax import lax
from jax.experimental import pallas as pl
from jax.experimental.pallas import tpu as pltpu
from jax.experimental.pallas import tpu_sc as plsc

NROWS = 1000000
BATCH = 16384
DIM = 32
NUM_CORES = 2
NUM_SUBCORES = 16
NUM_WORKERS = NUM_CORES * NUM_SUBCORES
BPW = BATCH // NUM_WORKERS
CHUNK = 128
NCHUNK = BPW // CHUNK
LANES = 16
LINE = 128
RPL = LINE // DIM
SECT = 1 << 18
SECT_MASK = SECT - 1
SECT_SHIFT = 18
NLINES = SECT
TL = 2048
TGRID = NLINES // TL


TAIL_START = NROWS - TL
REMAP_FROM = (TAIL_START // TL + 1) * TL
TAIL_BLOCK = (REMAP_FROM - 3 * SECT) // TL + 1
REMAP_DELTA = TAIL_START - TAIL_BLOCK * TL
LAST_SAFE = (NROWS - TL) // TL


def _transpose_body(x0, x1, x2, x3, xt, y_ref):
    i = pl.program_id(0)
    for p, x in enumerate((x0, x1, x2)):
        y_ref[:, p * DIM:(p + 1) * DIM] = x[...].T
    y_ref[:, 3 * DIM:4 * DIM] = jnp.where(
        i == TAIL_BLOCK, xt[...].T, x3[...].T)


def _tc_relayout(table_t):
    tail_t = table_t[:, TAIL_START:]
    specs = [
        pl.BlockSpec((DIM, TL), functools.partial(
            lambda p, i: (0, (p * SECT) // TL + i), p))
        for p in range(3)
    ]
    specs.append(pl.BlockSpec(
        (DIM, TL), lambda i: (0, jnp.minimum((3 * SECT) // TL + i,
                                             LAST_SAFE))))
    specs.append(pl.BlockSpec((DIM, TL), lambda i: (0, 0)))
    return pl.pallas_call(
        _transpose_body,
        grid=(TGRID,),
        in_specs=specs,
        out_specs=pl.BlockSpec((TL, LINE), lambda i: (i, 0)),
        out_shape=jax.ShapeDtypeStruct((NLINES, LINE), jnp.float32),
    )(table_t, table_t, table_t, table_t, tail_t)


def _relmf_body(users_hbm, items_hbm, uemb_hbm, iemb_hbm,
                u_out, i_out, r_out,
                uidx_v, iidx_v, ulin_v, ilin_v,
                lines, u_cols, i_cols, r_v, sem):
    wid = lax.axis_index("s") * NUM_CORES + lax.axis_index("c")
    base = wid * BPW

    for j in range(NCHUNK):
        pltpu.sync_copy(users_hbm.at[pl.ds(base + j * CHUNK, CHUNK)],
                        uidx_v.at[j])
        pltpu.sync_copy(items_hbm.at[pl.ds(base + j * CHUNK, CHUNK)],
                        iidx_v.at[j])

    def lin(j, carry):
        for k in range(CHUNK // LANES):
            s = pl.ds(k * LANES, LANES)
            for idxv, linv in ((uidx_v, ulin_v), (iidx_v, ilin_v)):
                v = idxv[j, s]
                linv[j, s] = jnp.where(v >= REMAP_FROM,
                                       v - REMAP_DELTA, v & SECT_MASK)
        return carry

    lax.fori_loop(0, NCHUNK, lin, 0)

    rows0 = lax.iota(jnp.int32, LANES)

    def gather_lines(lin_ref, table_hbm):
        copies = [
            pltpu.async_copy(table_hbm.at[lin_ref.at[j]],
                             lines.at[pl.ds(j * CHUNK, CHUNK)], sem)
            for j in range(NCHUNK)
        ]
        for c in copies:
            c.wait()

    gather_lines(ulin_v, uemb_hbm)

    def u_group(g, carry):
        j, q0 = g // (CHUNK // LANES), (g % (CHUNK // LANES)) * LANES
        rows = rows0 + g * LANES
        s = pl.ds(pl.multiple_of(g * LANES, LANES), LANES)
        ofs = lax.shift_right_logical(
            uidx_v[j, pl.ds(q0, LANES)], SECT_SHIFT) * DIM
        for d in range(DIM):
            u_cols[d, s] = plsc.load_gather(lines, [rows, ofs + d])
        return carry

    lax.fori_loop(0, BPW // LANES, u_group, 0)

    gather_lines(ilin_v, iemb_hbm)

    def i_group(g, carry):
        j, q0 = g // (CHUNK // LANES), (g % (CHUNK // LANES)) * LANES
        rows = rows0 + g * LANES
        s = pl.ds(pl.multiple_of(g * LANES, LANES), LANES)
        ofs = lax.shift_right_logical(
            iidx_v[j, pl.ds(q0, LANES)], SECT_SHIFT) * DIM
        acc = jnp.zeros((LANES,), jnp.float32)
        for d in range(DIM):
            ic = plsc.load_gather(lines, [rows, ofs + d])
            i_cols[d, s] = ic
            acc = acc + u_cols[d, s] * ic
        r_v[s] = acc
        return carry

    lax.fori_loop(0, BPW // LANES, i_group, 0)

    for k in range(DIM // 8):
        pltpu.sync_copy(u_cols.at[pl.ds(k * 8, 8)],
                        u_out.at[pl.ds(k * 8, 8), pl.ds(base, BPW)])
        pltpu.sync_copy(i_cols.at[pl.ds(k * 8, 8)],
                        i_out.at[pl.ds(k * 8, 8), pl.ds(base, BPW)])
    pltpu.sync_copy(r_v, r_out.at[pl.ds(base, BPW)])


_relmf_sc = functools.partial(
    pl.kernel,
    out_type=(
        jax.ShapeDtypeStruct((DIM, BATCH), jnp.float32),
        jax.ShapeDtypeStruct((DIM, BATCH), jnp.float32),
        jax.ShapeDtypeStruct((BATCH,), jnp.float32),
    ),
    mesh=plsc.VectorSubcoreMesh(core_axis_name="c", subcore_axis_name="s"),
    compiler_params=pltpu.CompilerParams(needs_layout_passes=False),
    scratch_types=[
        pltpu.VMEM((NCHUNK, CHUNK), jnp.int32),
        pltpu.VMEM((NCHUNK, CHUNK), jnp.int32),
        pltpu.VMEM((NCHUNK, CHUNK), jnp.int32),
        pltpu.VMEM((NCHUNK, CHUNK), jnp.int32),
        pltpu.VMEM((BPW, LINE), jnp.float32),
        pltpu.VMEM((DIM, BPW), jnp.float32),
        pltpu.VMEM((DIM, BPW), jnp.float32),
        pltpu.VMEM((BPW,), jnp.float32),
        pltpu.SemaphoreType.DMA,
    ],
)(_relmf_body)


def kernel(users, items, user_embeddings, item_embeddings):
    u_tab = _tc_relayout(user_embeddings.T)
    i_tab = _tc_relayout(item_embeddings.T)
    u_t, i_t, r_hats = _relmf_sc(users, items, u_tab, i_tab)
    return (u_t.T, i_t.T, r_hats)
